# Initial kernel scaffold; baseline (speedup 1.0000x reference)
import jax
import jax.numpy as jnp
from jax.experimental import pallas as pl


def kernel(x, W1, b1, W2, b2, Wr, br):
    raise NotImplementedError("write your pallas kernel here")


# speedup vs baseline: 3.0597x; 3.0597x over previous
import jax
import jax.numpy as jnp
from jax.experimental import pallas as pl
from jax.experimental.pallas import tpu as pltpu

E, D, F, T = 8, 768, 2048, 2048
CT = 512


def _moe_kernel(x_ref, W1_ref, b1_ref, W2_ref, b2_ref, Wr_ref, br_ref,
                out_ref, w_ref):
    e = pl.program_id(0)

    @pl.when(e == 0)
    def _():
        xb = x_ref[...].astype(jnp.bfloat16)
        logits = jnp.dot(xb, Wr_ref[...].astype(jnp.bfloat16),
                         preferred_element_type=jnp.float32) + br_ref[...]
        m = jnp.max(logits, axis=-1, keepdims=True)
        p = jnp.exp(logits - m)
        w_ref[...] = p / jnp.sum(p, axis=-1, keepdims=True)

    w1b = W1_ref[0].astype(jnp.bfloat16)
    w2b = W2_ref[0].astype(jnp.bfloat16)
    b1 = b1_ref[0]
    b2 = b2_ref[0]

    lane = jax.lax.broadcasted_iota(jnp.int32, (CT, E), 1)

    for i in range(T // CT):
        sl = pl.ds(i * CT, CT)
        xc = x_ref[sl, :].astype(jnp.bfloat16)
        h = jnp.dot(xc, w1b, preferred_element_type=jnp.float32) + b1
        h = 0.5 * h * (1.0 + jax.lax.erf(h * 0.7071067811865476))
        wc = jnp.sum(jnp.where(lane == e, w_ref[sl, :], 0.0), axis=1,
                     keepdims=True)
        hb = (h * wc).astype(jnp.bfloat16)
        contrib = jnp.dot(hb, w2b, preferred_element_type=jnp.float32) \
            + wc * b2

        @pl.when(e == 0)
        def _():
            out_ref[sl, :] = contrib

        @pl.when(e != 0)
        def _():
            out_ref[sl, :] = out_ref[sl, :] + contrib


def kernel(x, W1, b1, W2, b2, Wr, br):
    br2 = br.reshape(1, E)
    b1 = b1.reshape(E, 1, F)
    b2 = b2.reshape(E, 1, D)
    return pl.pallas_call(
        _moe_kernel,
        grid=(E,),
        in_specs=[
            pl.BlockSpec((T, D), lambda e: (0, 0)),
            pl.BlockSpec((1, D, F), lambda e: (e, 0, 0)),
            pl.BlockSpec((1, 1, F), lambda e: (e, 0, 0)),
            pl.BlockSpec((1, F, D), lambda e: (e, 0, 0)),
            pl.BlockSpec((1, 1, D), lambda e: (e, 0, 0)),
            pl.BlockSpec((D, E), lambda e: (0, 0)),
            pl.BlockSpec((1, E), lambda e: (0, 0)),
        ],
        out_specs=pl.BlockSpec((T, D), lambda e: (0, 0)),
        out_shape=jax.ShapeDtypeStruct((T, D), jnp.float32),
        scratch_shapes=[pltpu.VMEM((T, E), jnp.float32)],
        compiler_params=pltpu.CompilerParams(
            dimension_semantics=("arbitrary",),
        ),
    )(x, W1, b1, W2, b2, Wr, br2)
